# baseline (speedup 1.0000x reference)
import functools

import jax
import jax.numpy as jnp
from jax import lax
from jax.experimental import pallas as pl
from jax.experimental.pallas import tpu as pltpu
from jax.experimental.pallas import tpu_sc as plsc

N = 4096
NUMEL = N * N
HIGH_NUM = int(NUMEL * 0.01)
K_LOW = HIGH_NUM // 2
K_HIGH = NUMEL - HIGH_NUM // 2

NC, NS = 2, 16
NW = NC * NS
ROWS_W = N // NW
SLAB_R, SLAB_C = 8, 2048
SLAB = SLAB_R * SLAB_C
NSLAB = ROWS_W * N // SLAB
VGROUP = 16
NGROUP = SLAB // (16 * VGROUP)


def _zero_hist(hist_v, nwords):
    zero16 = jnp.zeros((16,), jnp.int32)

    def zbody(i, _):
        for t in range(8):
            hist_v[pl.ds(i * 128 + t * 16, 16)] = zero16
        return 0

    lax.fori_loop(0, nwords // 128, zbody, 0)


def _lane_reduce(hist_v, nbins):
    def rbody(i, _):
        acc = hist_v[pl.ds(i * 16, 16)]
        for l in range(1, 16):
            acc = acc + hist_v[pl.ds(l * nbins + i * 16, 16)]
        hist_v[pl.ds(i * 16, 16)] = acc
        return 0

    lax.fori_loop(0, nbins // 16, rbody, 0)


def _slab_at(x_hbm, row_base, s):
    r0 = row_base + lax.shift_right_logical(s, 1) * SLAB_R
    c0 = (s & 1) * SLAB_C
    return x_hbm.at[pl.ds(r0, SLAB_R), pl.ds(c0, SLAB_C)]


def _double_buffered_scan(x_hbm, row_base, slab0, slab1, sem0, sem1, process):
    pltpu.async_copy(_slab_at(x_hbm, row_base, 0), slab0, sem0)
    pltpu.async_copy(_slab_at(x_hbm, row_base, 1), slab1, sem1)

    def outer(g, _):
        for half, (slab, sem) in enumerate(((slab0, sem0), (slab1, sem1))):
            pltpu.make_async_copy(
                _slab_at(x_hbm, row_base, 0), slab, sem
            ).wait()
            lax.fori_loop(
                0, NGROUP, lambda j, c, s=slab: (process(j, s), 0)[1], 0
            )
            nxt = 2 * g + 2 + half

            @pl.when(nxt < NSLAB)
            def _():
                pltpu.async_copy(_slab_at(x_hbm, row_base, nxt), slab, sem)

        return 0

    lax.fori_loop(0, NSLAB // 2, outer, 0)


def _group_loads(slab, j):
    r = lax.shift_right_logical(j, 3)
    cg = (j & 7) * 256
    vs = [slab[r, pl.ds(cg + t * 16, 16)] for t in range(VGROUP)]
    return [lax.bitcast_convert_type(v, jnp.int32) for v in vs]


@functools.lru_cache(maxsize=None)
def _sc_kernels():
    mesh = plsc.VectorSubcoreMesh(
        core_axis_name="c", subcore_axis_name="s", num_cores=NC, num_subcores=NS
    )

    @functools.partial(
        pl.kernel,
        out_type=jax.ShapeDtypeStruct((NW, 4096), jnp.int32),
        mesh=mesh,
        scratch_types=[
            pltpu.VMEM((SLAB_R, SLAB_C), jnp.float32),
            pltpu.VMEM((SLAB_R, SLAB_C), jnp.float32),
            pltpu.VMEM((16 * 4096,), jnp.int32),
            pltpu.SemaphoreType.DMA,
            pltpu.SemaphoreType.DMA,
        ],
        compiler_params=pltpu.CompilerParams(needs_layout_passes=False),
    )
    def _hist_a(x_hbm, out_hbm, slab0, slab1, hist_v, sem0, sem1):
        cid = lax.axis_index("c")
        sid = lax.axis_index("s")
        wid = sid * NC + cid
        row_base = wid * ROWS_W

        _zero_hist(hist_v, 16 * 4096)

        lane_off = lax.iota(jnp.int32, 16) * 4096 + 2048
        ones = jnp.ones((16,), jnp.int32)

        def process(j, slab):
            us = _group_loads(slab, j)
            addrs = [lax.shift_right_arithmetic(u, 20) + lane_off for u in us]
            for t in range(VGROUP):
                plsc.addupdate_scatter(hist_v, [addrs[t]], ones)

        _double_buffered_scan(
            x_hbm, row_base, slab0, slab1, sem0, sem1, process
        )
        _lane_reduce(hist_v, 4096)
        pltpu.sync_copy(hist_v.at[pl.ds(0, 4096)], out_hbm.at[wid])

    def _make_hist_bc(shift_pref, shift_bin):
        @functools.partial(
            pl.kernel,
            out_type=jax.ShapeDtypeStruct((NW, 2048), jnp.int32),
            mesh=mesh,
            scratch_types=[
                pltpu.VMEM((SLAB_R, SLAB_C), jnp.float32),
                pltpu.VMEM((SLAB_R, SLAB_C), jnp.float32),
                pltpu.VMEM((16 * 2048,), jnp.int32),
                pltpu.VMEM((16,), jnp.int32),
                pltpu.VMEM((16,), jnp.int32),
                pltpu.SemaphoreType.DMA,
                pltpu.SemaphoreType.DMA,
            ],
            compiler_params=pltpu.CompilerParams(needs_layout_passes=False),
        )
        def _hist_bc(
            x_hbm, params_hbm, out_hbm, slab0, slab1, hist_v, plo_v, phi_v,
            sem0, sem1,
        ):
            cid = lax.axis_index("c")
            sid = lax.axis_index("s")
            wid = sid * NC + cid
            row_base = wid * ROWS_W

            pltpu.sync_copy(params_hbm.at[0, pl.ds(0, 16)], plo_v)
            pltpu.sync_copy(params_hbm.at[2, pl.ds(0, 16)], phi_v)
            pref_lo = plo_v[...]
            pref_hi = phi_v[...]

            _zero_hist(hist_v, 16 * 2048)

            lane_off = lax.iota(jnp.int32, 16) * 2048
            ones = jnp.ones((16,), jnp.int32)

            def process(j, slab):
                us = _group_loads(slab, j)
                prefs = [
                    lax.shift_right_arithmetic(u, shift_pref) for u in us
                ]
                if shift_bin:
                    bins = [
                        lax.shift_right_arithmetic(u, shift_bin) & 1023
                        for u in us
                    ]
                else:
                    bins = [u & 1023 for u in us]
                addrs = [b + lane_off for b in bins]
                m_lo = [p == pref_lo for p in prefs]
                m_hi = [p == pref_hi for p in prefs]
                for t in range(VGROUP):
                    plsc.addupdate_scatter(
                        hist_v, [addrs[t]], ones, mask=m_lo[t]
                    )
                    plsc.addupdate_scatter(
                        hist_v, [addrs[t] + 1024], ones, mask=m_hi[t]
                    )

            _double_buffered_scan(
                x_hbm, row_base, slab0, slab1, sem0, sem1, process
            )
            _lane_reduce(hist_v, 2048)
            pltpu.sync_copy(hist_v.at[pl.ds(0, 2048)], out_hbm.at[wid])

        return _hist_bc

    return _hist_a, _make_hist_bc(20, 10), _make_hist_bc(10, 0)


def _search(h2, k, descending):
    R = h2.shape[0]
    rt = jnp.sum(h2, axis=1, keepdims=True)
    ii = lax.broadcasted_iota(jnp.int32, (R, R), 0)
    jj = lax.broadcasted_iota(jnp.int32, (R, R), 1)
    rt_b = jnp.broadcast_to(rt, (R, R))
    cmp_row = (ii > jj) if descending else (ii < jj)
    rc = jnp.sum(jnp.where(cmp_row, rt_b, 0), axis=0, keepdims=True)
    rt_t = jnp.sum(jnp.where(ii == jj, rt_b, 0), axis=0, keepdims=True)
    sel_r = rc + rt_t < k
    m_rows = jnp.sum(sel_r.astype(jnp.int32))
    rstar = (R - 1 - m_rows) if descending else m_rows
    below_rows = jnp.sum(jnp.where(sel_r, rt_t, 0))

    rows_i = lax.broadcasted_iota(jnp.int32, (R, 128), 0)
    rowv = jnp.sum(jnp.where(rows_i == rstar, h2, 0), axis=0, keepdims=True)

    i128 = lax.broadcasted_iota(jnp.int32, (128, 128), 0)
    j128 = lax.broadcasted_iota(jnp.int32, (128, 128), 1)
    row_b = jnp.broadcast_to(rowv, (128, 128))
    cmp_lane = (j128 >= i128) if descending else (j128 <= i128)
    wc = jnp.sum(jnp.where(cmp_lane, row_b, 0), axis=1, keepdims=True)
    row_t = jnp.sum(jnp.where(i128 == j128, row_b, 0), axis=1, keepdims=True)
    sel_c = below_rows + wc < k
    m_l = jnp.sum(sel_c.astype(jnp.int32))
    cstar = (127 - m_l) if descending else m_l
    below = below_rows + jnp.sum(jnp.where(sel_c, row_t, 0))
    return rstar * 128 + cstar, below


def _search_signed(h2, k, pref):
    i_asc, b_asc = _search(h2, k, False)
    i_dsc, b_dsc = _search(h2, k, True)
    use_desc = pref < 0
    idx = jnp.where(use_desc, i_dsc, i_asc)
    below = jnp.where(use_desc, b_dsc, b_asc)
    return idx, k - below


def _pack_params(a, b, c, d):
    rows = lax.broadcasted_iota(jnp.int32, (8, 128), 0)
    z = jnp.zeros((8, 128), jnp.int32)
    out = jnp.where(rows == 0, a, z)
    out = jnp.where(rows == 1, b, out)
    out = jnp.where(rows == 2, c, out)
    return jnp.where(rows == 3, d, out)


def _ext_a_body(hist_ref, out_ref):
    acc = hist_ref[0]
    for w in range(1, NW):
        acc = acc + hist_ref[w]
    rows_i = lax.broadcasted_iota(jnp.int32, (32, 128), 0)
    hneg = jnp.where(rows_i < 16, acc, 0)
    hpos = acc - hneg
    neg_total = jnp.sum(hneg)

    def pick(k):
        i_n, b_n = _search(hneg, k, True)
        i_p, b_p = _search(hpos, k - neg_total, False)
        use_neg = k <= neg_total
        idx = jnp.where(use_neg, i_n, i_p)
        below = jnp.where(use_neg, b_n, b_p + neg_total)
        return idx - 2048, k - below

    p_lo, k_lo = pick(K_LOW)
    p_hi, k_hi = pick(K_HIGH)
    out_ref[...] = _pack_params(p_lo, k_lo, p_hi, k_hi)


def _ext_bc_body(hist_ref, params_ref, out_ref):
    acc_lo = hist_ref[0]
    acc_hi = hist_ref[1]
    for w in range(1, NW):
        acc_lo = acc_lo + hist_ref[2 * w]
        acc_hi = acc_hi + hist_ref[2 * w + 1]
    p_lo = params_ref[0, 0]
    k_lo = params_ref[1, 0]
    p_hi = params_ref[2, 0]
    k_hi = params_ref[3, 0]
    i_lo, kr_lo = _search_signed(acc_lo, k_lo, p_lo)
    i_hi, kr_hi = _search_signed(acc_hi, k_hi, p_hi)
    out_ref[...] = _pack_params(
        p_lo * 1024 + i_lo, kr_lo, p_hi * 1024 + i_hi, kr_hi
    )


def _ext_c_body(hist_ref, params_ref, out_ref):
    acc_lo = hist_ref[0]
    acc_hi = hist_ref[1]
    for w in range(1, NW):
        acc_lo = acc_lo + hist_ref[2 * w]
        acc_hi = acc_hi + hist_ref[2 * w + 1]
    p_lo = params_ref[0, 0]
    k_lo = params_ref[1, 0]
    p_hi = params_ref[2, 0]
    k_hi = params_ref[3, 0]
    i_lo, _ = _search_signed(acc_lo, k_lo, p_lo)
    i_hi, _ = _search_signed(acc_hi, k_hi, p_hi)
    u = _pack_params(p_lo * 1024 + i_lo, p_hi * 1024 + i_hi, 0, 0)
    out_ref[...] = lax.bitcast_convert_type(u, jnp.float32)


_ext_a = pl.pallas_call(
    _ext_a_body, out_shape=jax.ShapeDtypeStruct((8, 128), jnp.int32)
)
_ext_b = pl.pallas_call(
    _ext_bc_body,
    in_specs=[
        pl.BlockSpec(),
        pl.BlockSpec(memory_space=pltpu.SMEM),
    ],
    out_shape=jax.ShapeDtypeStruct((8, 128), jnp.int32),
)
_ext_c = pl.pallas_call(
    _ext_c_body,
    in_specs=[
        pl.BlockSpec(),
        pl.BlockSpec(memory_space=pltpu.SMEM),
    ],
    out_shape=jax.ShapeDtypeStruct((8, 128), jnp.float32),
)


ROWS_BLK = 256


def _quant_body(thr_ref, x_ref, out_ref):
    t_lo = thr_ref[0, 0]
    t_hi = thr_ref[1, 0]
    xb = x_ref[...]
    m = (xb > t_lo) & (xb < t_hi)
    low = jnp.where(m, xb, 0.0)
    high = xb - low

    def params(part, maxq):
        pmin = jnp.minimum(jnp.min(part, axis=1, keepdims=True), 0.0)
        pmax = jnp.maximum(jnp.max(part, axis=1, keepdims=True), 0.0)
        deg = (pmin == 0.0) & (pmax == 0.0)
        pmin = jnp.where(deg, -1.0, pmin)
        pmax = jnp.where(deg, 1.0, pmax)
        scale = (pmax - pmin) / maxq
        zero = jnp.round(-pmin / scale)
        return scale, zero

    l_scale, l_zero = params(low, 3.0)
    h_scale, h_zero = params(high, 255.0)
    l_inv = 1.0 / l_scale
    h_inv = 1.0 / h_scale
    ql = jnp.clip(jnp.round(low * l_inv) + l_zero, 0.0, 3.0)
    qh = jnp.clip(jnp.round(high * h_inv) + h_zero, 0.0, 255.0)
    out_ref[...] = l_scale * (ql - l_zero) + h_scale * (qh - h_zero)


_quant = pl.pallas_call(
    _quant_body,
    grid=(N // ROWS_BLK,),
    in_specs=[
        pl.BlockSpec(memory_space=pltpu.SMEM),
        pl.BlockSpec((ROWS_BLK, N), lambda i: (i, 0)),
    ],
    out_specs=pl.BlockSpec((ROWS_BLK, N), lambda i: (i, 0)),
    out_shape=jax.ShapeDtypeStruct((N, N), jnp.float32),
)


def kernel(x):
    _hist_a, _hist_b, _hist_c = _sc_kernels()
    hist_a = _hist_a(x)
    p_a = _ext_a(hist_a.reshape(NW, 32, 128))
    hist_b = _hist_b(x, p_a)
    p_b = _ext_b(hist_b.reshape(2 * NW, 8, 128), p_a)
    hist_c = _hist_c(x, p_b)
    thr = _ext_c(hist_c.reshape(2 * NW, 8, 128), p_b)
    return _quant(thr, x)


# speedup vs baseline: 1.0517x; 1.0517x over previous
import functools

import jax
import jax.numpy as jnp
from jax import lax
from jax.experimental import pallas as pl
from jax.experimental.pallas import tpu as pltpu
from jax.experimental.pallas import tpu_sc as plsc

N = 4096
NUMEL = N * N
HIGH_NUM = int(NUMEL * 0.01)
K_LOW = HIGH_NUM // 2
K_HIGH = NUMEL - HIGH_NUM // 2

NC, NS = 2, 16
NW = NC * NS
ROWS_W = N // NW
SLAB_R, SLAB_C = 8, 2048
SLAB = SLAB_R * SLAB_C
NSLAB = ROWS_W * N // SLAB
VGROUP = 16
NGROUP = SLAB // (16 * VGROUP)
CAPL = 1024


def _zero_hist(hist_v, nwords):
    zero16 = jnp.zeros((16,), jnp.int32)

    def zbody(i, _):
        for t in range(8):
            hist_v[pl.ds(i * 128 + t * 16, 16)] = zero16
        return 0

    lax.fori_loop(0, nwords // 128, zbody, 0)


def _lane_reduce(hist_v, nbins):
    def rbody(i, _):
        acc = hist_v[pl.ds(i * 16, 16)]
        for l in range(1, 16):
            acc = acc + hist_v[pl.ds(l * nbins + i * 16, 16)]
        hist_v[pl.ds(i * 16, 16)] = acc
        return 0

    lax.fori_loop(0, nbins // 16, rbody, 0)


def _slab_at(x_hbm, row_base, s):
    r0 = row_base + lax.shift_right_logical(s, 1) * SLAB_R
    c0 = (s & 1) * SLAB_C
    return x_hbm.at[pl.ds(r0, SLAB_R), pl.ds(c0, SLAB_C)]


def _double_buffered_scan(
    x_hbm, row_base, slab0, slab1, sem0, sem1, process, carry_init=0
):
    pltpu.async_copy(_slab_at(x_hbm, row_base, 0), slab0, sem0)
    pltpu.async_copy(_slab_at(x_hbm, row_base, 1), slab1, sem1)

    def outer(g, carry):
        for half, (slab, sem) in enumerate(((slab0, sem0), (slab1, sem1))):
            pltpu.make_async_copy(
                _slab_at(x_hbm, row_base, 0), slab, sem
            ).wait()
            carry = lax.fori_loop(
                0, NGROUP, lambda j, c, s=slab: process(j, s, c), carry
            )
            nxt = 2 * g + 2 + half

            @pl.when(nxt < NSLAB)
            def _():
                pltpu.async_copy(_slab_at(x_hbm, row_base, nxt), slab, sem)

        return carry

    return lax.fori_loop(0, NSLAB // 2, outer, carry_init)


def _group_loads(slab, j):
    r = lax.shift_right_logical(j, 3)
    cg = (j & 7) * 256
    vs = [slab[r, pl.ds(cg + t * 16, 16)] for t in range(VGROUP)]
    return [lax.bitcast_convert_type(v, jnp.int32) for v in vs]


@functools.lru_cache(maxsize=None)
def _sc_kernels():
    mesh = plsc.VectorSubcoreMesh(
        core_axis_name="c", subcore_axis_name="s", num_cores=NC, num_subcores=NS
    )

    @functools.partial(
        pl.kernel,
        out_type=jax.ShapeDtypeStruct((NW, 4096), jnp.int32),
        mesh=mesh,
        scratch_types=[
            pltpu.VMEM((SLAB_R, SLAB_C), jnp.float32),
            pltpu.VMEM((SLAB_R, SLAB_C), jnp.float32),
            pltpu.VMEM((16 * 4096,), jnp.int32),
            pltpu.SemaphoreType.DMA,
            pltpu.SemaphoreType.DMA,
        ],
        compiler_params=pltpu.CompilerParams(needs_layout_passes=False),
    )
    def _hist_a(x_hbm, out_hbm, slab0, slab1, hist_v, sem0, sem1):
        cid = lax.axis_index("c")
        sid = lax.axis_index("s")
        wid = sid * NC + cid
        row_base = wid * ROWS_W

        _zero_hist(hist_v, 16 * 4096)

        lane_off = lax.iota(jnp.int32, 16) * 4096 + 2048
        ones = jnp.ones((16,), jnp.int32)

        def process(j, slab, c):
            us = _group_loads(slab, j)
            addrs = [lax.shift_right_arithmetic(u, 20) + lane_off for u in us]
            for t in range(VGROUP):
                plsc.addupdate_scatter(hist_v, [addrs[t]], ones)
            return c

        _double_buffered_scan(
            x_hbm, row_base, slab0, slab1, sem0, sem1, process
        )
        _lane_reduce(hist_v, 4096)
        pltpu.sync_copy(hist_v.at[pl.ds(0, 4096)], out_hbm.at[wid])

    @functools.partial(
        pl.kernel,
        out_type=(
            jax.ShapeDtypeStruct((NW, 2048), jnp.int32),
            jax.ShapeDtypeStruct((NW, 16 * CAPL), jnp.int32),
            jax.ShapeDtypeStruct((NW, 16), jnp.int32),
        ),
        mesh=mesh,
        scratch_types=[
            pltpu.VMEM((SLAB_R, SLAB_C), jnp.float32),
            pltpu.VMEM((SLAB_R, SLAB_C), jnp.float32),
            pltpu.VMEM((16 * 2048,), jnp.int32),
            pltpu.VMEM((16 * CAPL,), jnp.int32),
            pltpu.VMEM((16,), jnp.int32),
            pltpu.VMEM((16,), jnp.int32),
            pltpu.VMEM((16,), jnp.int32),
            pltpu.SemaphoreType.DMA,
            pltpu.SemaphoreType.DMA,
        ],
        compiler_params=pltpu.CompilerParams(needs_layout_passes=False),
    )
    def _hist_b(
        x_hbm, params_hbm, out_hbm, coll_hbm, cnt_hbm, slab0, slab1, hist_v,
        coll_v, plo_v, phi_v, cnt_s, sem0, sem1,
    ):
        cid = lax.axis_index("c")
        sid = lax.axis_index("s")
        wid = sid * NC + cid
        row_base = wid * ROWS_W

        pltpu.sync_copy(params_hbm.at[0, pl.ds(0, 16)], plo_v)
        pltpu.sync_copy(params_hbm.at[2, pl.ds(0, 16)], phi_v)
        pref_lo = plo_v[...]
        pref_hi = phi_v[...]

        _zero_hist(hist_v, 16 * 2048)

        w1 = lax.shift_left(pref_lo + 1, 20)
        p1 = lax.shift_right_arithmetic(w1, 20)
        sent = jnp.where(p1 == pref_hi, lax.shift_left(pref_lo + 2, 20), w1)

        def fbody(i, _):
            for t in range(8):
                coll_v[pl.ds(i * 128 + t * 16, 16)] = sent
            return 0

        lax.fori_loop(0, (16 * CAPL) // 128, fbody, 0)

        lane_off = lax.iota(jnp.int32, 16) * 2048
        lane_cap = lax.iota(jnp.int32, 16) * CAPL
        ones = jnp.ones((16,), jnp.int32)

        def process(j, slab, cnt):
            us = _group_loads(slab, j)
            prefs = [lax.shift_right_arithmetic(u, 20) for u in us]
            addrs = [
                (lax.shift_right_arithmetic(u, 10) & 1023) + lane_off
                for u in us
            ]
            m_lo = [p == pref_lo for p in prefs]
            m_hi = [p == pref_hi for p in prefs]
            for t in range(VGROUP):
                plsc.addupdate_scatter(hist_v, [addrs[t]], ones, mask=m_lo[t])
                plsc.addupdate_scatter(
                    hist_v, [addrs[t] + 1024], ones, mask=m_hi[t]
                )
                m_any = m_lo[t] | m_hi[t]
                slot = lane_cap + jnp.minimum(cnt, CAPL - 1)
                plsc.store_scatter(coll_v, [slot], us[t], mask=m_any)
                cnt = cnt + m_any.astype(jnp.int32)
            return cnt

        cnt = _double_buffered_scan(
            x_hbm, row_base, slab0, slab1, sem0, sem1, process,
            jnp.zeros((16,), jnp.int32),
        )
        cnt_s[...] = cnt
        pltpu.sync_copy(cnt_s, cnt_hbm.at[wid])
        pltpu.sync_copy(coll_v, coll_hbm.at[wid])
        _lane_reduce(hist_v, 2048)
        pltpu.sync_copy(hist_v.at[pl.ds(0, 2048)], out_hbm.at[wid])

    def _bc_process(hist_v, lane_off, ones, pref_lo, pref_hi, shift_pref,
                    shift_bin, us):
        prefs = [lax.shift_right_arithmetic(u, shift_pref) for u in us]
        if shift_bin:
            bins = [
                lax.shift_right_arithmetic(u, shift_bin) & 1023 for u in us
            ]
        else:
            bins = [u & 1023 for u in us]
        addrs = [b + lane_off for b in bins]
        m_lo = [p == pref_lo for p in prefs]
        m_hi = [p == pref_hi for p in prefs]
        for t in range(VGROUP):
            plsc.addupdate_scatter(hist_v, [addrs[t]], ones, mask=m_lo[t])
            plsc.addupdate_scatter(
                hist_v, [addrs[t] + 1024], ones, mask=m_hi[t]
            )

    @functools.partial(
        pl.kernel,
        out_type=jax.ShapeDtypeStruct((NW, 2048), jnp.int32),
        mesh=mesh,
        scratch_types=[
            pltpu.VMEM((SLAB_R, SLAB_C), jnp.float32),
            pltpu.VMEM((SLAB_R, SLAB_C), jnp.float32),
            pltpu.VMEM((16 * 2048,), jnp.int32),
            pltpu.VMEM((16,), jnp.int32),
            pltpu.VMEM((16,), jnp.int32),
            pltpu.SemaphoreType.DMA,
            pltpu.SemaphoreType.DMA,
        ],
        compiler_params=pltpu.CompilerParams(needs_layout_passes=False),
    )
    def _hist_c_full(
        x_hbm, params_hbm, out_hbm, slab0, slab1, hist_v, plo_v, phi_v,
        sem0, sem1,
    ):
        cid = lax.axis_index("c")
        sid = lax.axis_index("s")
        wid = sid * NC + cid
        row_base = wid * ROWS_W

        pltpu.sync_copy(params_hbm.at[0, pl.ds(0, 16)], plo_v)
        pltpu.sync_copy(params_hbm.at[2, pl.ds(0, 16)], phi_v)
        pref_lo = plo_v[...]
        pref_hi = phi_v[...]

        _zero_hist(hist_v, 16 * 2048)

        lane_off = lax.iota(jnp.int32, 16) * 2048
        ones = jnp.ones((16,), jnp.int32)

        def process(j, slab, c):
            _bc_process(
                hist_v, lane_off, ones, pref_lo, pref_hi, 10, 0,
                _group_loads(slab, j),
            )
            return c

        _double_buffered_scan(
            x_hbm, row_base, slab0, slab1, sem0, sem1, process
        )
        _lane_reduce(hist_v, 2048)
        pltpu.sync_copy(hist_v.at[pl.ds(0, 2048)], out_hbm.at[wid])

    @functools.partial(
        pl.kernel,
        out_type=jax.ShapeDtypeStruct((NW, 2048), jnp.int32),
        mesh=mesh,
        scratch_types=[
            pltpu.VMEM((16 * CAPL,), jnp.int32),
            pltpu.VMEM((16 * 2048,), jnp.int32),
            pltpu.VMEM((16,), jnp.int32),
            pltpu.VMEM((16,), jnp.int32),
        ],
        compiler_params=pltpu.CompilerParams(needs_layout_passes=False),
    )
    def _hist_c_small(
        coll_hbm, params_hbm, out_hbm, coll_v, hist_v, plo_v, phi_v
    ):
        cid = lax.axis_index("c")
        sid = lax.axis_index("s")
        wid = sid * NC + cid

        pltpu.sync_copy(params_hbm.at[0, pl.ds(0, 16)], plo_v)
        pltpu.sync_copy(params_hbm.at[2, pl.ds(0, 16)], phi_v)
        pref_lo = plo_v[...]
        pref_hi = phi_v[...]

        _zero_hist(hist_v, 16 * 2048)
        pltpu.sync_copy(coll_hbm.at[wid], coll_v)

        lane_off = lax.iota(jnp.int32, 16) * 2048
        ones = jnp.ones((16,), jnp.int32)

        def body(j, c):
            us = [
                coll_v[pl.ds(j * 16 * VGROUP + t * 16, 16)]
                for t in range(VGROUP)
            ]
            _bc_process(
                hist_v, lane_off, ones, pref_lo, pref_hi, 10, 0, us
            )
            return c

        lax.fori_loop(0, (16 * CAPL) // (16 * VGROUP), body, 0)
        _lane_reduce(hist_v, 2048)
        pltpu.sync_copy(hist_v.at[pl.ds(0, 2048)], out_hbm.at[wid])

    return _hist_a, _hist_b, _hist_c_full, _hist_c_small


def _search(h2, k, descending):
    R = h2.shape[0]
    rt = jnp.sum(h2, axis=1, keepdims=True)
    ii = lax.broadcasted_iota(jnp.int32, (R, R), 0)
    jj = lax.broadcasted_iota(jnp.int32, (R, R), 1)
    rt_b = jnp.broadcast_to(rt, (R, R))
    cmp_row = (ii > jj) if descending else (ii < jj)
    rc = jnp.sum(jnp.where(cmp_row, rt_b, 0), axis=0, keepdims=True)
    rt_t = jnp.sum(jnp.where(ii == jj, rt_b, 0), axis=0, keepdims=True)
    sel_r = rc + rt_t < k
    m_rows = jnp.sum(sel_r.astype(jnp.int32))
    rstar = (R - 1 - m_rows) if descending else m_rows
    below_rows = jnp.sum(jnp.where(sel_r, rt_t, 0))

    rows_i = lax.broadcasted_iota(jnp.int32, (R, 128), 0)
    rowv = jnp.sum(jnp.where(rows_i == rstar, h2, 0), axis=0, keepdims=True)

    i128 = lax.broadcasted_iota(jnp.int32, (128, 128), 0)
    j128 = lax.broadcasted_iota(jnp.int32, (128, 128), 1)
    row_b = jnp.broadcast_to(rowv, (128, 128))
    cmp_lane = (j128 >= i128) if descending else (j128 <= i128)
    wc = jnp.sum(jnp.where(cmp_lane, row_b, 0), axis=1, keepdims=True)
    row_t = jnp.sum(jnp.where(i128 == j128, row_b, 0), axis=1, keepdims=True)
    sel_c = below_rows + wc < k
    m_l = jnp.sum(sel_c.astype(jnp.int32))
    cstar = (127 - m_l) if descending else m_l
    below = below_rows + jnp.sum(jnp.where(sel_c, row_t, 0))
    return rstar * 128 + cstar, below


def _search_signed(h2, k, pref):
    i_asc, b_asc = _search(h2, k, False)
    i_dsc, b_dsc = _search(h2, k, True)
    use_desc = pref < 0
    idx = jnp.where(use_desc, i_dsc, i_asc)
    below = jnp.where(use_desc, b_dsc, b_asc)
    return idx, k - below


def _pack_params(a, b, c, d):
    rows = lax.broadcasted_iota(jnp.int32, (8, 128), 0)
    z = jnp.zeros((8, 128), jnp.int32)
    out = jnp.where(rows == 0, a, z)
    out = jnp.where(rows == 1, b, out)
    out = jnp.where(rows == 2, c, out)
    return jnp.where(rows == 3, d, out)


def _ext_a_body(hist_ref, out_ref):
    acc = hist_ref[0]
    for w in range(1, NW):
        acc = acc + hist_ref[w]
    rows_i = lax.broadcasted_iota(jnp.int32, (32, 128), 0)
    hneg = jnp.where(rows_i < 16, acc, 0)
    hpos = acc - hneg
    neg_total = jnp.sum(hneg)

    def pick(k):
        i_n, b_n = _search(hneg, k, True)
        i_p, b_p = _search(hpos, k - neg_total, False)
        use_neg = k <= neg_total
        idx = jnp.where(use_neg, i_n, i_p)
        below = jnp.where(use_neg, b_n, b_p + neg_total)
        return idx - 2048, k - below

    p_lo, k_lo = pick(K_LOW)
    p_hi, k_hi = pick(K_HIGH)
    out_ref[...] = _pack_params(p_lo, k_lo, p_hi, k_hi)


def _ext_bc_body(hist_ref, params_ref, out_ref):
    acc_lo = hist_ref[0]
    acc_hi = hist_ref[1]
    for w in range(1, NW):
        acc_lo = acc_lo + hist_ref[2 * w]
        acc_hi = acc_hi + hist_ref[2 * w + 1]
    p_lo = params_ref[0, 0]
    k_lo = params_ref[1, 0]
    p_hi = params_ref[2, 0]
    k_hi = params_ref[3, 0]
    i_lo, kr_lo = _search_signed(acc_lo, k_lo, p_lo)
    i_hi, kr_hi = _search_signed(acc_hi, k_hi, p_hi)
    out_ref[...] = _pack_params(
        p_lo * 1024 + i_lo, kr_lo, p_hi * 1024 + i_hi, kr_hi
    )


def _ext_c_body(hist_ref, params_ref, out_ref):
    acc_lo = hist_ref[0]
    acc_hi = hist_ref[1]
    for w in range(1, NW):
        acc_lo = acc_lo + hist_ref[2 * w]
        acc_hi = acc_hi + hist_ref[2 * w + 1]
    p_lo = params_ref[0, 0]
    k_lo = params_ref[1, 0]
    p_hi = params_ref[2, 0]
    k_hi = params_ref[3, 0]
    i_lo, _ = _search_signed(acc_lo, k_lo, p_lo)
    i_hi, _ = _search_signed(acc_hi, k_hi, p_hi)
    u = _pack_params(p_lo * 1024 + i_lo, p_hi * 1024 + i_hi, 0, 0)
    out_ref[...] = lax.bitcast_convert_type(u, jnp.float32)


_ext_a = pl.pallas_call(
    _ext_a_body, out_shape=jax.ShapeDtypeStruct((8, 128), jnp.int32)
)
_ext_b = pl.pallas_call(
    _ext_bc_body,
    in_specs=[
        pl.BlockSpec(),
        pl.BlockSpec(memory_space=pltpu.SMEM),
    ],
    out_shape=jax.ShapeDtypeStruct((8, 128), jnp.int32),
)
_ext_c = pl.pallas_call(
    _ext_c_body,
    in_specs=[
        pl.BlockSpec(),
        pl.BlockSpec(memory_space=pltpu.SMEM),
    ],
    out_shape=jax.ShapeDtypeStruct((8, 128), jnp.float32),
)


ROWS_BLK = 256


def _quant_body(thr_ref, x_ref, out_ref):
    t_lo = thr_ref[0, 0]
    t_hi = thr_ref[1, 0]
    xb = x_ref[...]
    m = (xb > t_lo) & (xb < t_hi)
    low = jnp.where(m, xb, 0.0)
    high = xb - low

    def params(part, maxq):
        pmin = jnp.minimum(jnp.min(part, axis=1, keepdims=True), 0.0)
        pmax = jnp.maximum(jnp.max(part, axis=1, keepdims=True), 0.0)
        deg = (pmin == 0.0) & (pmax == 0.0)
        pmin = jnp.where(deg, -1.0, pmin)
        pmax = jnp.where(deg, 1.0, pmax)
        scale = (pmax - pmin) / maxq
        zero = jnp.round(-pmin / scale)
        return scale, zero

    l_scale, l_zero = params(low, 3.0)
    h_scale, h_zero = params(high, 255.0)
    ql = jnp.clip(jnp.round(low / l_scale) + l_zero, 0.0, 3.0)
    qh = jnp.clip(jnp.round(high / h_scale) + h_zero, 0.0, 255.0)
    out_ref[...] = l_scale * (ql - l_zero) + h_scale * (qh - h_zero)


_quant = pl.pallas_call(
    _quant_body,
    grid=(N // ROWS_BLK,),
    in_specs=[
        pl.BlockSpec(memory_space=pltpu.SMEM),
        pl.BlockSpec((ROWS_BLK, N), lambda i: (i, 0)),
    ],
    out_specs=pl.BlockSpec((ROWS_BLK, N), lambda i: (i, 0)),
    out_shape=jax.ShapeDtypeStruct((N, N), jnp.float32),
)


def kernel(x):
    _hist_a, _hist_b, _hist_c_full, _hist_c_small = _sc_kernels()
    hist_a = _hist_a(x)
    p_a = _ext_a(hist_a.reshape(NW, 32, 128))
    hist_b, coll, cnts = _hist_b(x, p_a)
    p_b = _ext_b(hist_b.reshape(2 * NW, 8, 128), p_a)
    overflow = jnp.any(cnts > CAPL)
    hist_c = lax.cond(
        overflow,
        lambda: _hist_c_full(x, p_b),
        lambda: _hist_c_small(coll, p_b),
    )
    thr = _ext_c(hist_c.reshape(2 * NW, 8, 128), p_b)
    return _quant(thr, x)


# speedup vs baseline: 1.4007x; 1.3319x over previous
import functools

import jax
import jax.numpy as jnp
from jax import lax
from jax.experimental import pallas as pl
from jax.experimental.pallas import tpu as pltpu
from jax.experimental.pallas import tpu_sc as plsc

N = 4096
NUMEL = N * N
HIGH_NUM = int(NUMEL * 0.01)
K_LOW = HIGH_NUM // 2
K_HIGH = NUMEL - HIGH_NUM // 2

NC, NS = 2, 16
NW = NC * NS
ROWS_W = N // NW
SLAB_R, SLAB_C = 8, 2048
SLAB = SLAB_R * SLAB_C
NSLAB = ROWS_W * N // SLAB
VGROUP = 16
NGROUP = SLAB // (16 * VGROUP)
CAPL = 1024


def _zero_hist(hist_v, nwords):
    zero16 = jnp.zeros((16,), jnp.int32)

    def zbody(i, _):
        for t in range(8):
            hist_v[pl.ds(i * 128 + t * 16, 16)] = zero16
        return 0

    lax.fori_loop(0, nwords // 128, zbody, 0)


def _lane_reduce(hist_v, nbins):
    def rbody(i, _):
        acc = hist_v[pl.ds(i * 16, 16)]
        for l in range(1, 16):
            acc = acc + hist_v[pl.ds(l * nbins + i * 16, 16)]
        hist_v[pl.ds(i * 16, 16)] = acc
        return 0

    lax.fori_loop(0, nbins // 16, rbody, 0)


def _slab_at(x_hbm, row_base, s):
    r0 = row_base + lax.shift_right_logical(s, 1) * SLAB_R
    c0 = (s & 1) * SLAB_C
    return x_hbm.at[pl.ds(r0, SLAB_R), pl.ds(c0, SLAB_C)]


def _double_buffered_scan(
    x_hbm, row_base, slab0, slab1, sem0, sem1, process, carry_init=0
):
    pltpu.async_copy(_slab_at(x_hbm, row_base, 0), slab0, sem0)
    pltpu.async_copy(_slab_at(x_hbm, row_base, 1), slab1, sem1)

    def outer(g, carry):
        for half, (slab, sem) in enumerate(((slab0, sem0), (slab1, sem1))):
            pltpu.make_async_copy(
                _slab_at(x_hbm, row_base, 0), slab, sem
            ).wait()
            carry = lax.fori_loop(
                0, NGROUP, lambda j, c, s=slab: process(j, s, c), carry
            )
            nxt = 2 * g + 2 + half

            @pl.when(nxt < NSLAB)
            def _():
                pltpu.async_copy(_slab_at(x_hbm, row_base, nxt), slab, sem)

        return carry

    return lax.fori_loop(0, NSLAB // 2, outer, carry_init)


def _group_loads(slab, j):
    r = lax.shift_right_logical(j, 3)
    cg = (j & 7) * 256
    vs = [slab[r, pl.ds(cg + t * 16, 16)] for t in range(VGROUP)]
    return [lax.bitcast_convert_type(v, jnp.int32) for v in vs]


@functools.lru_cache(maxsize=None)
def _sc_kernels():
    mesh = plsc.VectorSubcoreMesh(
        core_axis_name="c", subcore_axis_name="s", num_cores=NC, num_subcores=NS
    )

    @functools.partial(
        pl.kernel,
        out_type=jax.ShapeDtypeStruct((NW, 4096), jnp.int32),
        mesh=mesh,
        scratch_types=[
            pltpu.VMEM((SLAB_R, SLAB_C), jnp.float32),
            pltpu.VMEM((SLAB_R, SLAB_C), jnp.float32),
            pltpu.VMEM((16 * 4096,), jnp.int32),
            pltpu.SemaphoreType.DMA,
            pltpu.SemaphoreType.DMA,
        ],
        compiler_params=pltpu.CompilerParams(needs_layout_passes=False),
    )
    def _hist_a(x_hbm, out_hbm, slab0, slab1, hist_v, sem0, sem1):
        cid = lax.axis_index("c")
        sid = lax.axis_index("s")
        wid = sid * NC + cid
        row_base = wid * ROWS_W

        _zero_hist(hist_v, 16 * 4096)

        lane_off = lax.iota(jnp.int32, 16) * 4096 + 2048
        ones = jnp.ones((16,), jnp.int32)

        def process(j, slab, c):
            us = _group_loads(slab, j)
            addrs = [lax.shift_right_arithmetic(u, 20) + lane_off for u in us]
            for t in range(VGROUP):
                plsc.addupdate_scatter(hist_v, [addrs[t]], ones)
            return c

        _double_buffered_scan(
            x_hbm, row_base, slab0, slab1, sem0, sem1, process
        )
        _lane_reduce(hist_v, 4096)
        pltpu.sync_copy(hist_v.at[pl.ds(0, 4096)], out_hbm.at[wid])

    @functools.partial(
        pl.kernel,
        out_type=(
            jax.ShapeDtypeStruct((NW, 16 * CAPL), jnp.int32),
            jax.ShapeDtypeStruct((NW, 16), jnp.int32),
        ),
        mesh=mesh,
        scratch_types=[
            pltpu.VMEM((SLAB_R, SLAB_C), jnp.float32),
            pltpu.VMEM((SLAB_R, SLAB_C), jnp.float32),
            pltpu.VMEM((16 * CAPL,), jnp.int32),
            pltpu.VMEM((16,), jnp.int32),
            pltpu.VMEM((16,), jnp.int32),
            pltpu.VMEM((16,), jnp.int32),
            pltpu.SemaphoreType.DMA,
            pltpu.SemaphoreType.DMA,
        ],
        compiler_params=pltpu.CompilerParams(needs_layout_passes=False),
    )
    def _collect_b(
        x_hbm, params_hbm, coll_hbm, cnt_hbm, slab0, slab1,
        coll_v, plo_v, phi_v, cnt_s, sem0, sem1,
    ):
        cid = lax.axis_index("c")
        sid = lax.axis_index("s")
        wid = sid * NC + cid
        row_base = wid * ROWS_W

        pltpu.sync_copy(params_hbm.at[0, pl.ds(0, 16)], plo_v)
        pltpu.sync_copy(params_hbm.at[2, pl.ds(0, 16)], phi_v)
        pref_lo = plo_v[...]
        pref_hi = phi_v[...]

        w1 = lax.shift_left(pref_lo + 1, 20)
        p1 = lax.shift_right_arithmetic(w1, 20)
        sent = jnp.where(p1 == pref_hi, lax.shift_left(pref_lo + 2, 20), w1)

        def fbody(i, _):
            for t in range(8):
                coll_v[pl.ds(i * 128 + t * 16, 16)] = sent
            return 0

        lax.fori_loop(0, (16 * CAPL) // 128, fbody, 0)

        lane_cap = lax.iota(jnp.int32, 16) * CAPL

        def process(j, slab, cnt):
            us = _group_loads(slab, j)
            prefs = [lax.shift_right_arithmetic(u, 20) for u in us]
            m_any = [
                (p == pref_lo) | (p == pref_hi) for p in prefs
            ]
            for t in range(VGROUP):
                slot = lane_cap + jnp.minimum(cnt, CAPL - 1)
                plsc.store_scatter(coll_v, [slot], us[t], mask=m_any[t])
                cnt = cnt + m_any[t].astype(jnp.int32)
            return cnt

        cnt = _double_buffered_scan(
            x_hbm, row_base, slab0, slab1, sem0, sem1, process,
            jnp.zeros((16,), jnp.int32),
        )
        cnt_s[...] = cnt
        pltpu.sync_copy(cnt_s, cnt_hbm.at[wid])
        pltpu.sync_copy(coll_v, coll_hbm.at[wid])

    def _bc_process(hist_v, lane_off, ones, pref_lo, pref_hi, shift_pref,
                    shift_bin, us):
        prefs = [lax.shift_right_arithmetic(u, shift_pref) for u in us]
        if shift_bin:
            bins = [
                lax.shift_right_arithmetic(u, shift_bin) & 1023 for u in us
            ]
        else:
            bins = [u & 1023 for u in us]
        addrs = [b + lane_off for b in bins]
        m_lo = [p == pref_lo for p in prefs]
        m_hi = [p == pref_hi for p in prefs]
        for t in range(VGROUP):
            plsc.addupdate_scatter(hist_v, [addrs[t]], ones, mask=m_lo[t])
            plsc.addupdate_scatter(
                hist_v, [addrs[t] + 1024], ones, mask=m_hi[t]
            )

    def _make_full(shift_pref, shift_bin):
        @functools.partial(
            pl.kernel,
            out_type=jax.ShapeDtypeStruct((NW, 2048), jnp.int32),
            mesh=mesh,
            scratch_types=[
                pltpu.VMEM((SLAB_R, SLAB_C), jnp.float32),
                pltpu.VMEM((SLAB_R, SLAB_C), jnp.float32),
                pltpu.VMEM((16 * 2048,), jnp.int32),
                pltpu.VMEM((16,), jnp.int32),
                pltpu.VMEM((16,), jnp.int32),
                pltpu.SemaphoreType.DMA,
                pltpu.SemaphoreType.DMA,
            ],
            compiler_params=pltpu.CompilerParams(needs_layout_passes=False),
        )
        def _hist_full(
            x_hbm, params_hbm, out_hbm, slab0, slab1, hist_v, plo_v, phi_v,
            sem0, sem1,
        ):
            cid = lax.axis_index("c")
            sid = lax.axis_index("s")
            wid = sid * NC + cid
            row_base = wid * ROWS_W

            pltpu.sync_copy(params_hbm.at[0, pl.ds(0, 16)], plo_v)
            pltpu.sync_copy(params_hbm.at[2, pl.ds(0, 16)], phi_v)
            pref_lo = plo_v[...]
            pref_hi = phi_v[...]

            _zero_hist(hist_v, 16 * 2048)

            lane_off = lax.iota(jnp.int32, 16) * 2048
            ones = jnp.ones((16,), jnp.int32)

            def process(j, slab, c):
                _bc_process(
                    hist_v, lane_off, ones, pref_lo, pref_hi, shift_pref,
                    shift_bin, _group_loads(slab, j),
                )
                return c

            _double_buffered_scan(
                x_hbm, row_base, slab0, slab1, sem0, sem1, process
            )
            _lane_reduce(hist_v, 2048)
            pltpu.sync_copy(hist_v.at[pl.ds(0, 2048)], out_hbm.at[wid])

        return _hist_full

    def _make_small(shift_pref, shift_bin):
        @functools.partial(
            pl.kernel,
            out_type=jax.ShapeDtypeStruct((NW, 2048), jnp.int32),
            mesh=mesh,
            scratch_types=[
                pltpu.VMEM((16 * CAPL,), jnp.int32),
                pltpu.VMEM((16 * 2048,), jnp.int32),
                pltpu.VMEM((16,), jnp.int32),
                pltpu.VMEM((16,), jnp.int32),
            ],
            compiler_params=pltpu.CompilerParams(needs_layout_passes=False),
        )
        def _hist_small(
            coll_hbm, params_hbm, out_hbm, coll_v, hist_v, plo_v, phi_v
        ):
            cid = lax.axis_index("c")
            sid = lax.axis_index("s")
            wid = sid * NC + cid

            pltpu.sync_copy(params_hbm.at[0, pl.ds(0, 16)], plo_v)
            pltpu.sync_copy(params_hbm.at[2, pl.ds(0, 16)], phi_v)
            pref_lo = plo_v[...]
            pref_hi = phi_v[...]

            _zero_hist(hist_v, 16 * 2048)
            pltpu.sync_copy(coll_hbm.at[wid], coll_v)

            lane_off = lax.iota(jnp.int32, 16) * 2048
            ones = jnp.ones((16,), jnp.int32)

            def body(j, c):
                us = [
                    coll_v[pl.ds(j * 16 * VGROUP + t * 16, 16)]
                    for t in range(VGROUP)
                ]
                _bc_process(
                    hist_v, lane_off, ones, pref_lo, pref_hi, shift_pref,
                    shift_bin, us,
                )
                return c

            lax.fori_loop(0, (16 * CAPL) // (16 * VGROUP), body, 0)
            _lane_reduce(hist_v, 2048)
            pltpu.sync_copy(hist_v.at[pl.ds(0, 2048)], out_hbm.at[wid])

        return _hist_small

    return (
        _hist_a,
        _collect_b,
        _make_full(20, 10),
        _make_small(20, 10),
        _make_full(10, 0),
        _make_small(10, 0),
    )


def _search(h2, k, descending):
    R = h2.shape[0]
    rt = jnp.sum(h2, axis=1, keepdims=True)
    ii = lax.broadcasted_iota(jnp.int32, (R, R), 0)
    jj = lax.broadcasted_iota(jnp.int32, (R, R), 1)
    rt_b = jnp.broadcast_to(rt, (R, R))
    cmp_row = (ii > jj) if descending else (ii < jj)
    rc = jnp.sum(jnp.where(cmp_row, rt_b, 0), axis=0, keepdims=True)
    rt_t = jnp.sum(jnp.where(ii == jj, rt_b, 0), axis=0, keepdims=True)
    sel_r = rc + rt_t < k
    m_rows = jnp.sum(sel_r.astype(jnp.int32))
    rstar = (R - 1 - m_rows) if descending else m_rows
    below_rows = jnp.sum(jnp.where(sel_r, rt_t, 0))

    rows_i = lax.broadcasted_iota(jnp.int32, (R, 128), 0)
    rowv = jnp.sum(jnp.where(rows_i == rstar, h2, 0), axis=0, keepdims=True)

    i128 = lax.broadcasted_iota(jnp.int32, (128, 128), 0)
    j128 = lax.broadcasted_iota(jnp.int32, (128, 128), 1)
    row_b = jnp.broadcast_to(rowv, (128, 128))
    cmp_lane = (j128 >= i128) if descending else (j128 <= i128)
    wc = jnp.sum(jnp.where(cmp_lane, row_b, 0), axis=1, keepdims=True)
    row_t = jnp.sum(jnp.where(i128 == j128, row_b, 0), axis=1, keepdims=True)
    sel_c = below_rows + wc < k
    m_l = jnp.sum(sel_c.astype(jnp.int32))
    cstar = (127 - m_l) if descending else m_l
    below = below_rows + jnp.sum(jnp.where(sel_c, row_t, 0))
    return rstar * 128 + cstar, below


def _search_signed(h2, k, pref):
    i_asc, b_asc = _search(h2, k, False)
    i_dsc, b_dsc = _search(h2, k, True)
    use_desc = pref < 0
    idx = jnp.where(use_desc, i_dsc, i_asc)
    below = jnp.where(use_desc, b_dsc, b_asc)
    return idx, k - below


def _pack_params(a, b, c, d):
    rows = lax.broadcasted_iota(jnp.int32, (8, 128), 0)
    z = jnp.zeros((8, 128), jnp.int32)
    out = jnp.where(rows == 0, a, z)
    out = jnp.where(rows == 1, b, out)
    out = jnp.where(rows == 2, c, out)
    return jnp.where(rows == 3, d, out)


def _ext_a_body(hist_ref, out_ref):
    acc = hist_ref[0]
    for w in range(1, NW):
        acc = acc + hist_ref[w]
    rows_i = lax.broadcasted_iota(jnp.int32, (32, 128), 0)
    hneg = jnp.where(rows_i < 16, acc, 0)
    hpos = acc - hneg
    neg_total = jnp.sum(hneg)

    def pick(k):
        i_n, b_n = _search(hneg, k, True)
        i_p, b_p = _search(hpos, k - neg_total, False)
        use_neg = k <= neg_total
        idx = jnp.where(use_neg, i_n, i_p)
        below = jnp.where(use_neg, b_n, b_p + neg_total)
        return idx - 2048, k - below

    p_lo, k_lo = pick(K_LOW)
    p_hi, k_hi = pick(K_HIGH)
    out_ref[...] = _pack_params(p_lo, k_lo, p_hi, k_hi)


def _ext_bc_body(hist_ref, params_ref, out_ref):
    acc_lo = hist_ref[0]
    acc_hi = hist_ref[1]
    for w in range(1, NW):
        acc_lo = acc_lo + hist_ref[2 * w]
        acc_hi = acc_hi + hist_ref[2 * w + 1]
    p_lo = params_ref[0, 0]
    k_lo = params_ref[1, 0]
    p_hi = params_ref[2, 0]
    k_hi = params_ref[3, 0]
    i_lo, kr_lo = _search_signed(acc_lo, k_lo, p_lo)
    i_hi, kr_hi = _search_signed(acc_hi, k_hi, p_hi)
    out_ref[...] = _pack_params(
        p_lo * 1024 + i_lo, kr_lo, p_hi * 1024 + i_hi, kr_hi
    )


def _ext_c_body(hist_ref, params_ref, out_ref):
    acc_lo = hist_ref[0]
    acc_hi = hist_ref[1]
    for w in range(1, NW):
        acc_lo = acc_lo + hist_ref[2 * w]
        acc_hi = acc_hi + hist_ref[2 * w + 1]
    p_lo = params_ref[0, 0]
    k_lo = params_ref[1, 0]
    p_hi = params_ref[2, 0]
    k_hi = params_ref[3, 0]
    i_lo, _ = _search_signed(acc_lo, k_lo, p_lo)
    i_hi, _ = _search_signed(acc_hi, k_hi, p_hi)
    u = _pack_params(p_lo * 1024 + i_lo, p_hi * 1024 + i_hi, 0, 0)
    out_ref[...] = lax.bitcast_convert_type(u, jnp.float32)


_ext_a = pl.pallas_call(
    _ext_a_body, out_shape=jax.ShapeDtypeStruct((8, 128), jnp.int32)
)
_ext_b = pl.pallas_call(
    _ext_bc_body,
    in_specs=[
        pl.BlockSpec(),
        pl.BlockSpec(memory_space=pltpu.SMEM),
    ],
    out_shape=jax.ShapeDtypeStruct((8, 128), jnp.int32),
)
_ext_c = pl.pallas_call(
    _ext_c_body,
    in_specs=[
        pl.BlockSpec(),
        pl.BlockSpec(memory_space=pltpu.SMEM),
    ],
    out_shape=jax.ShapeDtypeStruct((8, 128), jnp.float32),
)


ROWS_BLK = 256


def _quant_body(thr_ref, x_ref, out_ref):
    t_lo = thr_ref[0, 0]
    t_hi = thr_ref[1, 0]
    xb = x_ref[...]
    m = (xb > t_lo) & (xb < t_hi)
    low = jnp.where(m, xb, 0.0)
    high = xb - low

    def params(part, maxq):
        pmin = jnp.minimum(jnp.min(part, axis=1, keepdims=True), 0.0)
        pmax = jnp.maximum(jnp.max(part, axis=1, keepdims=True), 0.0)
        deg = (pmin == 0.0) & (pmax == 0.0)
        pmin = jnp.where(deg, -1.0, pmin)
        pmax = jnp.where(deg, 1.0, pmax)
        scale = (pmax - pmin) / maxq
        zero = jnp.round(-pmin / scale)
        return scale, zero

    l_scale, l_zero = params(low, 3.0)
    h_scale, h_zero = params(high, 255.0)
    ql = jnp.clip(jnp.round(low / l_scale) + l_zero, 0.0, 3.0)
    qh = jnp.clip(jnp.round(high / h_scale) + h_zero, 0.0, 255.0)
    out_ref[...] = l_scale * (ql - l_zero) + h_scale * (qh - h_zero)


_quant = pl.pallas_call(
    _quant_body,
    grid=(N // ROWS_BLK,),
    in_specs=[
        pl.BlockSpec(memory_space=pltpu.SMEM),
        pl.BlockSpec((ROWS_BLK, N), lambda i: (i, 0)),
    ],
    out_specs=pl.BlockSpec((ROWS_BLK, N), lambda i: (i, 0)),
    out_shape=jax.ShapeDtypeStruct((N, N), jnp.float32),
)


def kernel(x):
    (
        _hist_a,
        _collect_b,
        _hist_b_full,
        _hist_b_small,
        _hist_c_full,
        _hist_c_small,
    ) = _sc_kernels()
    hist_a = _hist_a(x)
    p_a = _ext_a(hist_a.reshape(NW, 32, 128))
    coll, cnts = _collect_b(x, p_a)
    overflow = jnp.any(cnts > CAPL)
    hist_b = lax.cond(
        overflow,
        lambda: _hist_b_full(x, p_a),
        lambda: _hist_b_small(coll, p_a),
    )
    p_b = _ext_b(hist_b.reshape(2 * NW, 8, 128), p_a)
    hist_c = lax.cond(
        overflow,
        lambda: _hist_c_full(x, p_b),
        lambda: _hist_c_small(coll, p_b),
    )
    thr = _ext_c(hist_c.reshape(2 * NW, 8, 128), p_b)
    return _quant(thr, x)
